# baseline (speedup 1.0000x reference)
import functools

import jax
import jax.numpy as jnp
from jax import lax
from jax.experimental import pallas as pl
from jax.experimental.pallas import tpu as pltpu
from jax.experimental.pallas import tpu_sc as plsc

TOKEN_NUMS = 100000
PAD_TOKEN = TOKEN_NUMS + 1
EMBED_DIM = 128
BZ, SL, T = 1024, 50, 20
HID = 256

NW = 32
SEG = BZ * T
SW = 56
NSEG_W = SEG // NW
CHS = 160

NID = BZ * SL + BZ
IDS_W = NID // NW
CH = 272

@functools.cache
def _build_sc_gather():
    mesh = plsc.VectorSubcoreMesh(core_axis_name="c", subcore_axis_name="s")
    return functools.partial(
        pl.kernel,
        mesh=mesh,
        out_type=jax.ShapeDtypeStruct((NID, EMBED_DIM), jnp.float32),
        scratch_types=[
            pltpu.VMEM((IDS_W,), jnp.int32),
            pltpu.VMEM((CH, EMBED_DIM), jnp.float32),
            pltpu.SemaphoreType.DMA,
        ],
    )(_sc_gather_body)


def _sc_gather_body(table_hbm, ids_hbm, out_hbm, idx_v, rows_v, sem):
    wid = lax.axis_index("s") * 2 + lax.axis_index("c")
    base = wid * IDS_W
    pltpu.sync_copy(ids_hbm.at[pl.ds(base, IDS_W)], idx_v)
    for c in range(IDS_W // CH):
        pltpu.async_copy(
            table_hbm.at[idx_v.at[pl.ds(c * CH, CH)]], rows_v, sem
        ).wait()
        pltpu.sync_copy(rows_v, out_hbm.at[pl.ds(base + c * CH, CH)])


@functools.cache
def _build_sc_agent_sum():
    mesh = plsc.VectorSubcoreMesh(core_axis_name="c", subcore_axis_name="s")
    return functools.partial(
        pl.kernel,
        mesh=mesh,
        out_type=jax.ShapeDtypeStruct((SEG, EMBED_DIM), jnp.float32),
        scratch_types=[
            pltpu.VMEM((CHS, SW), jnp.float32),
            pltpu.VMEM((CHS, SW), jnp.int32),
            pltpu.VMEM((2, SW, EMBED_DIM), jnp.float32),
            pltpu.VMEM((CHS, EMBED_DIM), jnp.float32),
            pltpu.SemaphoreType.DMA,
            pltpu.SemaphoreType.DMA,
        ],
    )(_sc_agent_sum_body)


def _sc_agent_sum_body(table_hbm, tok_hbm, out_hbm, tok_v, idx_v, rows_v,
                       outb_v, sem0, sem1):
    wid = lax.axis_index("s") * 2 + lax.axis_index("c")
    base = wid * NSEG_W

    def start(li, b):
        sem = sem0 if b == 0 else sem1
        pltpu.make_async_copy(
            table_hbm.at[idx_v.at[li]], rows_v.at[b], sem
        ).start()

    def wait(li, b):
        sem = sem0 if b == 0 else sem1
        pltpu.make_async_copy(
            table_hbm.at[idx_v.at[li]], rows_v.at[b], sem
        ).wait()

    def sum_rows(b, li):
        def srow(s, accs):
            return tuple(
                accs[c] + rows_v[b, s, pl.ds(c * 16, 16)] for c in range(8)
            )
        accs = lax.fori_loop(
            0, SW, srow,
            tuple(jnp.zeros((16,), jnp.float32) for _ in range(8)),
        )
        for c in range(8):
            outb_v[li, pl.ds(c * 16, 16)] = accs[c]

    for ch in range(NSEG_W // CHS):
        cb = ch * CHS
        pltpu.sync_copy(tok_hbm.at[pl.ds(base + cb, CHS)], tok_v)

        def idx_body(i, _):
            for c0 in (0, 16, 32, 40):
                t = tok_v[i, pl.ds(c0, 16)]
                ti = jnp.clip(t.astype(jnp.int32), 0, TOKEN_NUMS + 2)
                idx_v[i, pl.ds(c0, 16)] = jnp.where(t != -1.0, ti, PAD_TOKEN)
            return 0

        lax.fori_loop(0, CHS, idx_body, 0)

        start(0, 0)

        def pair(p, _):
            i0 = 2 * p
            wait(i0, 0)
            start(i0 + 1, 1)
            sum_rows(0, i0)
            wait(i0 + 1, 1)

            @pl.when(p + 1 < CHS // 2)
            def _():
                start(i0 + 2, 0)

            sum_rows(1, i0 + 1)
            return 0

        lax.fori_loop(0, CHS // 2, pair, 0)
        pltpu.sync_copy(outb_v, out_hbm.at[pl.ds(base + cb, CHS)])


def _tc_self(emb_cat, ego, w1a, w1b, b1, w2, b2):
    def body(emb_ref, ego_ref, w1a_ref, w1b_ref, b1_ref, w2_ref, b2_ref,
             out_ref):
        h = jnp.maximum(
            jnp.dot(emb_ref[:], w1a_ref[:], preferred_element_type=jnp.float32)
            + jnp.dot(ego_ref[:], w1b_ref[:],
                      preferred_element_type=jnp.float32)
            + b1_ref[:], 0.0)
        out_ref[:] = (
            jnp.dot(h, w2_ref[:], preferred_element_type=jnp.float32)
            + b2_ref[:])

    return pl.pallas_call(
        body,
        grid=(SL,),
        in_specs=[
            pl.BlockSpec((BZ, EMBED_DIM), lambda i: (i, 0)),
            pl.BlockSpec((BZ, 3), lambda i: (0, 0)),
            pl.BlockSpec((EMBED_DIM, HID), lambda i: (0, 0)),
            pl.BlockSpec((3, HID), lambda i: (0, 0)),
            pl.BlockSpec((1, HID), lambda i: (0, 0)),
            pl.BlockSpec((HID, EMBED_DIM), lambda i: (0, 0)),
            pl.BlockSpec((1, EMBED_DIM), lambda i: (0, 0)),
        ],
        out_specs=pl.BlockSpec((BZ, EMBED_DIM), lambda i: (i, 0)),
        out_shape=jax.ShapeDtypeStruct((BZ * SL, EMBED_DIM), jnp.float32),
        compiler_params=pltpu.CompilerParams(
            dimension_semantics=("parallel",)),
    )(emb_cat, ego, w1a, w1b, b1, w2, b2)


BB = 16
RPB = BB * T * SL
SEGB = BB * T


def _tc_env(af, asum, emb_cat, pad_row, wf9, w1p, w1g, b1, w2, b2):
    def body(af_ref, asum_ref, g_ref, pad_ref, wf_ref, w1p_ref, w1g_ref,
             b1_ref, w2_ref, b2_ref, out_ref):
        af_blk = af_ref[:]
        tok = af_blk[:, 0:1]
        m = (tok != -1.0).astype(jnp.float32)
        af9 = jnp.concatenate([af_blk * m, m], axis=1)
        f = jnp.maximum(
            jnp.dot(af9, wf_ref[:], preferred_element_type=jnp.float32), 0.0)
        fsum = jnp.sum(f.reshape(SEGB, SL, EMBED_DIM), axis=1)
        cnt = jnp.sum(m.reshape(SEGB, SL, 1), axis=1)
        esum = asum_ref[:] - (float(SW) - cnt) * pad_ref[:]
        pooled = (esum + fsum) / jnp.clip(cnt, 1.0, None)
        g2 = jnp.dot(g_ref[:], w1g_ref[:],
                     preferred_element_type=jnp.float32)
        g2b = jnp.broadcast_to(g2[:, None, :], (BB, T, HID)).reshape(SEGB, HID)
        hb = jnp.maximum(
            jnp.dot(pooled, w1p_ref[:], preferred_element_type=jnp.float32)
            + g2b + b1_ref[:], 0.0)
        out_ref[:] = (
            jnp.dot(hb, w2_ref[:], preferred_element_type=jnp.float32)
            + b2_ref[:])

    return pl.pallas_call(
        body,
        grid=(BZ // BB,),
        in_specs=[
            pl.BlockSpec((RPB, 8), lambda i: (i, 0)),
            pl.BlockSpec((SEGB, EMBED_DIM), lambda i: (i, 0)),
            pl.BlockSpec((BB, EMBED_DIM), lambda i: (BZ * SL // BB + i, 0)),
            pl.BlockSpec((1, EMBED_DIM), lambda i: (0, 0)),
            pl.BlockSpec((9, EMBED_DIM), lambda i: (0, 0)),
            pl.BlockSpec((EMBED_DIM, HID), lambda i: (0, 0)),
            pl.BlockSpec((EMBED_DIM, HID), lambda i: (0, 0)),
            pl.BlockSpec((1, HID), lambda i: (0, 0)),
            pl.BlockSpec((HID, EMBED_DIM), lambda i: (0, 0)),
            pl.BlockSpec((1, EMBED_DIM), lambda i: (0, 0)),
        ],
        out_specs=pl.BlockSpec((SEGB, EMBED_DIM), lambda i: (i, 0)),
        out_shape=jax.ShapeDtypeStruct((SEG, EMBED_DIM), jnp.float32),
        compiler_params=pltpu.CompilerParams(
            dimension_semantics=("parallel",)),
    )(af, asum, emb_cat, pad_row, wf9, w1p, w1g, b1, w2, b2)


def kernel(input_ids, ego_info, agent_info, goal, token_table,
           W_s1, b_s1, W_s2, b_s2, W_f, b_f, W_b1, b_b1, W_b2, b_b2):
    ids_cat = jnp.concatenate(
        [input_ids.reshape(-1), goal]).astype(jnp.int32)
    emb_cat = _build_sc_gather()(token_table, ids_cat)

    tok = agent_info[..., 0].reshape(SEG, SL)
    tok_p = jnp.concatenate(
        [tok, jnp.full((SEG, SW - SL), -1.0, tok.dtype)], axis=1)
    asum = _build_sc_agent_sum()(token_table, tok_p)

    af = agent_info.reshape(SEG * SL, 1 + W_f.shape[0])
    wf9 = jnp.concatenate(
        [jnp.zeros((1, EMBED_DIM), W_f.dtype), W_f, b_f[None]], axis=0)
    pad_row = token_table[PAD_TOKEN][None]

    self_flat = _tc_self(
        emb_cat, ego_info, W_s1[:EMBED_DIM], W_s1[EMBED_DIM:],
        b_s1[None], W_s2, b_s2[None])
    env_flat = _tc_env(
        af, asum, emb_cat, pad_row, wf9, W_b1[:EMBED_DIM], W_b1[EMBED_DIM:],
        b_b1[None], W_b2, b_b2[None])
    return (self_flat.reshape(BZ, SL, EMBED_DIM),
            env_flat.reshape(BZ, T, EMBED_DIM))
